# baseline (speedup 1.0000x reference)
import functools

import jax
import jax.numpy as jnp
import numpy as np
from jax import lax
from jax.experimental import pallas as pl
from jax.experimental.pallas import tpu as pltpu
from jax.experimental.pallas import tpu_sc as plsc

_BATCH = 65536
_N_PRED = 256
_COLS = (0, 3, 17, 42, 97, 128, 200, 255)
_SIGNS = (-1.0, 1.0, -1.0, 1.0, 1.0, -1.0, 1.0, -1.0)
_L = len(_COLS)
_MIN_W = 0.0
_MAX_W = 500.0

_LANES = 16
_NUM_CORES = 2
_NUM_SUBCORES = 16
_NW = _NUM_CORES * _NUM_SUBCORES
_RPW = _BATCH // _NW
_WPW = _RPW * _L
_STEPS = _RPW // _LANES

_IDX_CONST = np.asarray(_COLS, dtype=np.int32).reshape(-1, 1)
_GATHER_WORDS = (
    np.arange(_BATCH, dtype=np.int32)[:, None] * _N_PRED
    + np.asarray(_COLS, dtype=np.int32)[None, :]
).reshape(-1)


def _tec_body(ga_hbm, idx_hbm, w_hbm, out_hbm, idxv, colv, outv, wv, sem):
    wid = lax.axis_index("s") * _NUM_CORES + lax.axis_index("c")
    base = wid * _WPW

    pltpu.sync_copy(idx_hbm.at[pl.ds(base, _WPW)], idxv)
    gather = pltpu.make_async_copy(ga_hbm.at[idxv], colv, sem)
    gather.start()

    pltpu.sync_copy(w_hbm, wv)
    w16 = wv[...]
    w16 = jnp.minimum(jnp.maximum(w16, _MIN_W), _MAX_W)

    gather.wait()

    lane = lax.broadcasted_iota(jnp.int32, (_LANES,), 0)

    def step(i, carry):
        pos0 = i * (_LANES * _L)
        pidx = pos0 + lane * _L
        xs = []
        for j, s in enumerate(_SIGNS):
            x = plsc.load_gather(colv, [pidx + j])
            xs.append(-x if s < 0 else x)
        m = xs[0]
        for x in xs[1:]:
            m = jnp.maximum(m, x)
        es = [jnp.exp(x - m) for x in xs]
        tot = es[0]
        for e in es[1:]:
            tot = tot + e
        scale = w16 / tot
        for j, s in enumerate(_SIGNS):
            d = es[j] * scale
            if s < 0:
                d = -d
            plsc.store_scatter(outv, [pidx + j], d)
        return carry

    lax.fori_loop(0, _STEPS, step, 0)

    pltpu.sync_copy(outv, out_hbm.at[pl.ds(base, _WPW)])


@jax.jit
def _delta_sc(ga_flat, gather_words, wvec):
    mesh = plsc.VectorSubcoreMesh(core_axis_name="c", subcore_axis_name="s")
    k = functools.partial(
        pl.kernel,
        mesh=mesh,
        compiler_params=pltpu.CompilerParams(
            use_tc_tiling_on_sc=False, needs_layout_passes=False),
        out_type=jax.ShapeDtypeStruct((_BATCH * _L,), jnp.float32),
        scratch_types=[
            pltpu.VMEM((_WPW,), jnp.int32),
            pltpu.VMEM((_WPW,), jnp.float32),
            pltpu.VMEM((_WPW,), jnp.float32),
            pltpu.VMEM((_LANES,), jnp.float32),
            pltpu.SemaphoreType.DMA,
        ],
    )(_tec_body)
    return k(ga_flat, gather_words, wvec)


def kernel(ground_atoms, clause_weight):
    wvec = jnp.broadcast_to(jnp.reshape(clause_weight, (1,)), (_LANES,))
    flat = _delta_sc(
        ground_atoms.reshape(-1), jnp.asarray(_GATHER_WORDS), wvec)
    return (flat.reshape(_BATCH, _L), jnp.asarray(_IDX_CONST))


# speedup vs baseline: 1.7409x; 1.7409x over previous
import functools

import jax
import jax.numpy as jnp
import numpy as np
from jax import lax
from jax.experimental import pallas as pl
from jax.experimental.pallas import tpu as pltpu
from jax.experimental.pallas import tpu_sc as plsc

_BATCH = 65536
_N_PRED = 256
_COLS = (0, 3, 17, 42, 97, 128, 200, 255)
_SIGNS = (-1.0, 1.0, -1.0, 1.0, 1.0, -1.0, 1.0, -1.0)
_L = len(_COLS)
_MIN_W = 0.0
_MAX_W = 500.0

_LANES = 16
_NUM_CORES = 2
_NUM_SUBCORES = 16
_NW = _NUM_CORES * _NUM_SUBCORES
_RPW = _BATCH // _NW
_CHUNK = 128
_NCHUNK = _RPW // _CHUNK
_GROUPS = _CHUNK // _LANES

_IDX_CONST = np.asarray(_COLS, dtype=np.int32).reshape(-1, 1)


def _tec_body(ga_hbm, w_hbm, out_hbm, sa, sb, ov, wv, sema, semb):
    wid = lax.axis_index("s") * _NUM_CORES + lax.axis_index("c")
    base = wid * _RPW

    pltpu.sync_copy(w_hbm, wv)
    w16 = wv[...]
    w16 = jnp.minimum(jnp.maximum(w16, _MIN_W), _MAX_W)

    lane = lax.broadcasted_iota(jnp.int32, (_LANES,), 0)
    cvecs = [jnp.full((_LANES,), c, jnp.int32) for c in _COLS]
    jvecs = [jnp.full((_LANES,), j, jnp.int32) for j in range(_L)]

    pltpu.make_async_copy(
        ga_hbm.at[pl.ds(base, _CHUNK)], sa, sema).start()
    pltpu.make_async_copy(
        ga_hbm.at[pl.ds(base + _CHUNK, _CHUNK)], sb, semb).start()

    def body(g, carry):
        for b, (buf, sem) in enumerate(((sa, sema), (sb, semb))):
            c = 2 * g + b
            row0 = base + c * _CHUNK
            pltpu.make_async_copy(
                ga_hbm.at[pl.ds(row0, _CHUNK)], buf, sem).wait()

            for s in range(_GROUPS):
                rowv = lane + (s * _LANES)
                xs = []
                for j, sg in enumerate(_SIGNS):
                    x = plsc.load_gather(buf, [rowv, cvecs[j]])
                    xs.append(-x if sg < 0 else x)
                m = xs[0]
                for x in xs[1:]:
                    m = jnp.maximum(m, x)
                es = [jnp.exp(x - m) for x in xs]
                tot = es[0]
                for e in es[1:]:
                    tot = tot + e
                scale = w16 / tot
                for j, sg in enumerate(_SIGNS):
                    d = es[j] * scale
                    if sg < 0:
                        d = -d
                    plsc.store_scatter(ov, [rowv, jvecs[j]], d)

            @pl.when(g < _NCHUNK // 2 - 1)
            def _prefetch():
                pltpu.make_async_copy(
                    ga_hbm.at[pl.ds(row0 + 2 * _CHUNK, _CHUNK)], buf,
                    sem).start()

            pltpu.sync_copy(ov, out_hbm.at[pl.ds(row0, _CHUNK)])
        return carry

    lax.fori_loop(0, _NCHUNK // 2, body, 0)


@jax.jit
def _delta_sc(ground_atoms, wvec):
    mesh = plsc.VectorSubcoreMesh(core_axis_name="c", subcore_axis_name="s")
    k = functools.partial(
        pl.kernel,
        mesh=mesh,
        compiler_params=pltpu.CompilerParams(needs_layout_passes=False),
        out_type=jax.ShapeDtypeStruct((_BATCH, _L), jnp.float32),
        scratch_types=[
            pltpu.VMEM((_CHUNK, _N_PRED), jnp.float32),
            pltpu.VMEM((_CHUNK, _N_PRED), jnp.float32),
            pltpu.VMEM((_CHUNK, _L), jnp.float32),
            pltpu.VMEM((_LANES,), jnp.float32),
            pltpu.SemaphoreType.DMA,
            pltpu.SemaphoreType.DMA,
        ],
    )(_tec_body)
    return k(ground_atoms, wvec)


def kernel(ground_atoms, clause_weight):
    wvec = jnp.broadcast_to(jnp.reshape(clause_weight, (1,)), (_LANES,))
    delta = _delta_sc(ground_atoms, wvec)
    return (delta, jnp.asarray(_IDX_CONST))


# speedup vs baseline: 2.4326x; 1.3973x over previous
import functools

import jax
import jax.numpy as jnp
import numpy as np
from jax import lax
from jax.experimental import pallas as pl
from jax.experimental.pallas import tpu as pltpu
from jax.experimental.pallas import tpu_sc as plsc

_BATCH = 65536
_N_PRED = 256
_COLS = (0, 3, 17, 42, 97, 128, 200, 255)
_SIGNS = (-1.0, 1.0, -1.0, 1.0, 1.0, -1.0, 1.0, -1.0)
_L = len(_COLS)
_MIN_W = 0.0
_MAX_W = 500.0

_LANES = 16
_NUM_CORES = 2
_NUM_SUBCORES = 16
_NW = _NUM_CORES * _NUM_SUBCORES
_RPW = _BATCH // _NW
_CHUNK = 128
_NCHUNK = _RPW // _CHUNK
_GROUPS = _CHUNK // _LANES

_IDX_CONST = np.asarray(_COLS, dtype=np.int32).reshape(-1, 1)


def _tec_body(ga_hbm, w_hbm, out_hbm, sa, sb, ov, wv, sema, semb):
    wid = lax.axis_index("s") * _NUM_CORES + lax.axis_index("c")
    base = wid * _RPW

    pltpu.sync_copy(w_hbm, wv)
    w16 = wv[...]
    w16 = jnp.minimum(jnp.maximum(w16, _MIN_W), _MAX_W)

    lane = lax.broadcasted_iota(jnp.int32, (_LANES,), 0)
    cvecs = [jnp.full((_LANES,), c, jnp.int32) for c in _COLS]
    jvecs = [jnp.full((_LANES,), j, jnp.int32) for j in range(_L)]

    pltpu.make_async_copy(
        ga_hbm.at[pl.ds(base, _CHUNK)], sa, sema).start()
    pltpu.make_async_copy(
        ga_hbm.at[pl.ds(base + _CHUNK, _CHUNK)], sb, semb).start()

    def body(g, carry):
        for b, (buf, sem) in enumerate(((sa, sema), (sb, semb))):
            c = 2 * g + b
            row0 = base + c * _CHUNK
            pltpu.make_async_copy(
                ga_hbm.at[pl.ds(row0, _CHUNK)], buf, sem).wait()

            loc0 = c * _CHUNK
            for s in range(_GROUPS):
                rowv = lane + (s * _LANES)
                xs = []
                for j, sg in enumerate(_SIGNS):
                    x = plsc.load_gather(buf, [rowv, cvecs[j]])
                    xs.append(-x if sg < 0 else x)
                m = xs[0]
                for x in xs[1:]:
                    m = jnp.maximum(m, x)
                es = [jnp.exp(x - m) for x in xs]
                tot = es[0]
                for e in es[1:]:
                    tot = tot + e
                scale = w16 / tot
                outrow = loc0 + s * _LANES + lane
                for j, sg in enumerate(_SIGNS):
                    d = es[j] * scale
                    if sg < 0:
                        d = -d
                    plsc.store_scatter(ov, [jvecs[j], outrow], d)

            @pl.when(g < _NCHUNK // 2 - 1)
            def _prefetch():
                pltpu.make_async_copy(
                    ga_hbm.at[pl.ds(row0 + 2 * _CHUNK, _CHUNK)], buf,
                    sem).start()
        return carry

    lax.fori_loop(0, _NCHUNK // 2, body, 0)
    pltpu.sync_copy(ov, out_hbm.at[:, pl.ds(base, _RPW)])


@jax.jit
def _delta_sc(ground_atoms, wvec):
    mesh = plsc.VectorSubcoreMesh(core_axis_name="c", subcore_axis_name="s")
    k = functools.partial(
        pl.kernel,
        mesh=mesh,
        compiler_params=pltpu.CompilerParams(needs_layout_passes=False),
        out_type=jax.ShapeDtypeStruct((_L, _BATCH), jnp.float32),
        scratch_types=[
            pltpu.VMEM((_CHUNK, _N_PRED), jnp.float32),
            pltpu.VMEM((_CHUNK, _N_PRED), jnp.float32),
            pltpu.VMEM((_L, _RPW), jnp.float32),
            pltpu.VMEM((_LANES,), jnp.float32),
            pltpu.SemaphoreType.DMA,
            pltpu.SemaphoreType.DMA,
        ],
    )(_tec_body)
    return k(ground_atoms, wvec)


def kernel(ground_atoms, clause_weight):
    wvec = jnp.broadcast_to(jnp.reshape(clause_weight, (1,)), (_LANES,))
    delta_t = _delta_sc(ground_atoms, wvec)
    return (delta_t.T, jnp.asarray(_IDX_CONST))
